# baseline (speedup 1.0000x reference)
import functools

import jax
import jax.numpy as jnp
from jax import lax
from jax.experimental import pallas as pl
from jax.experimental.pallas import tpu as pltpu
from jax.experimental.pallas import tpu_sc as plsc

N = 10000
E = 320000
D = 128

NC = 2
NS = 16
NW = NC * NS

SUB = 128
SUBE = 32
GRP = 16
EPAD = 327680
EPAD2 = 655360
RSPREAD = 512

NA = N + 8
RT0 = 624
RT15 = NA - 15 * RT0


@functools.cache
def _mesh():
    return plsc.VectorSubcoreMesh(core_axis_name="c", subcore_axis_name="s")


def _zero_rows(ref, nrows, width):
    def body(i, carry):
        for j in range(width // 16):
            ref[i, pl.ds(j * 16, 16)] = jnp.zeros((16,), jnp.float32)
        return carry
    lax.fori_loop(0, nrows, body, 0)


def _fill_ones(ref, nrows, width):
    def body(i, carry):
        for j in range(width // 16):
            ref[i, pl.ds(j * 16, 16)] = jnp.ones((16,), jnp.float32)
        return carry
    lax.fori_loop(0, nrows, body, 0)


def _tile_rows(s):
    r0 = s * RT0
    cnt_is_last = s == NS - 1
    return r0, cnt_is_last


def _zero_acc_slice(acc, zb, s, width):
    r0 = s * RT0
    def body(j, carry):
        pltpu.sync_copy(zb, acc.at[pl.ds(r0 + j * SUB, SUB)])
        return carry
    lax.fori_loop(0, 4, body, 0)

    @pl.when(s < NS - 1)
    def _():
        pltpu.sync_copy(zb.at[pl.ds(0, 112)], acc.at[pl.ds(r0 + 512, 112)])

    @pl.when(s == NS - 1)
    def _():
        pltpu.sync_copy(zb, acc.at[pl.ds(r0 + 512, SUB)])
        pltpu.sync_copy(zb.at[pl.ds(0, 8)], acc.at[pl.ds(r0 + 640, 8)])


def _export_acc_slice(acc, stage, out, c, s):
    r0 = s * RT0
    base = c * N + r0

    def body(j, carry):
        pltpu.sync_copy(acc.at[pl.ds(r0 + j * SUB, SUB)], stage)
        pltpu.sync_copy(stage, out.at[pl.ds(base + j * SUB, SUB)])
        return carry
    lax.fori_loop(0, 4, body, 0)

    @pl.when(s < NS - 1)
    def _():
        pltpu.sync_copy(acc.at[pl.ds(r0 + 512, 112)], stage.at[pl.ds(0, 112)])
        pltpu.sync_copy(stage.at[pl.ds(0, 112)], out.at[pl.ds(base + 512, 112)])

    @pl.when(s == NS - 1)
    def _():
        pltpu.sync_copy(acc.at[pl.ds(r0 + 512, SUB)], stage)
        pltpu.sync_copy(stage, out.at[pl.ds(base + 512, SUB)])


NBUF = 4


def _make_scatter_body(spw):
    ngrp = spw // GRP

    def _sc_scatter_body(g_hbm, src2d, dst2d, out,
                         isa, ida, isb, idb, b0, b1, b2, b3, st, acc,
                         g0s, g1s, g2s, g3s, c0s, c1s, c2s, c3s, ps, pd):
        c = lax.axis_index("c")
        s = lax.axis_index("s")
        wid = c * NS + s

        _zero_rows(st, SUB, D)
        _zero_acc_slice(acc, st, s, D)
        plsc.subcore_barrier()

        bufs = (b0, b1, b2, b3)
        gsem = (g0s, g1s, g2s, g3s)
        csem = (c0s, c1s, c2s, c3s)

        def group(j, cur, nxt, prefetch):
            csi, cdi = cur
            nsi, ndi = nxt
            pdescs = []
            if prefetch:
                nb = wid * spw + (j + 1) * GRP
                pdescs.append(
                    pltpu.async_copy(src2d.at[pl.ds(nb, GRP)], nsi, ps))
                pdescs.append(
                    pltpu.async_copy(dst2d.at[pl.ds(nb, GRP)], ndi, pd))
            gd = [None] * GRP
            sd = [None] * GRP
            for b in range(NBUF):
                gd[b] = pltpu.async_copy(g_hbm.at[csi.at[b]], bufs[b], gsem[b])
            for k in range(GRP):
                b = k % NBUF
                gd[k].wait()
                sd[k] = pltpu.async_copy(bufs[b], acc.at[cdi.at[k]],
                                         csem[b], add=True)
                if 2 <= k < GRP - 2:
                    sd[k - 2].wait()
                    b2_ = (k + 2) % NBUF
                    gd[k + 2] = pltpu.async_copy(
                        g_hbm.at[csi.at[k + 2]], bufs[b2_], gsem[b2_])
            for k in range(GRP - NBUF, GRP):
                sd[k].wait()
            for d in pdescs:
                d.wait()

        gb0 = wid * spw
        pltpu.sync_copy(src2d.at[pl.ds(gb0, GRP)], isa)
        pltpu.sync_copy(dst2d.at[pl.ds(gb0, GRP)], ida)

        ia = (isa, ida)
        ib = (isb, idb)
        group(0, ia, ib, True)

        def pair(p, carry):
            group(2 * p + 1, ib, ia, True)
            group(2 * p + 2, ia, ib, True)
            return carry
        lax.fori_loop(0, (ngrp - 2) // 2, pair, 0)

        group(ngrp - 1, ib, ia, False)

        plsc.subcore_barrier()
        _export_acc_slice(acc, st, out, c, s)

    return _sc_scatter_body


@functools.cache
def _sc_scatter_kernel(spw, nrows):
    return pl.kernel(
        _make_scatter_body(spw),
        out_type=jax.ShapeDtypeStruct((2 * N, D), jnp.float32),
        mesh=_mesh(),
        scratch_types=[
            pltpu.VMEM((GRP, SUBE), jnp.int32),
            pltpu.VMEM((GRP, SUBE), jnp.int32),
            pltpu.VMEM((GRP, SUBE), jnp.int32),
            pltpu.VMEM((GRP, SUBE), jnp.int32),
            pltpu.VMEM((SUBE, D), jnp.float32),
            pltpu.VMEM((SUBE, D), jnp.float32),
            pltpu.VMEM((SUBE, D), jnp.float32),
            pltpu.VMEM((SUBE, D), jnp.float32),
            pltpu.VMEM((SUB, D), jnp.float32),
            pltpu.VMEM_SHARED((NA, D), jnp.float32),
            pltpu.SemaphoreType.DMA,
            pltpu.SemaphoreType.DMA,
            pltpu.SemaphoreType.DMA,
            pltpu.SemaphoreType.DMA,
            pltpu.SemaphoreType.DMA,
            pltpu.SemaphoreType.DMA,
            pltpu.SemaphoreType.DMA,
            pltpu.SemaphoreType.DMA,
            pltpu.SemaphoreType.DMA,
            pltpu.SemaphoreType.DMA,
        ],
    )


def _sc_scatter(g_hbm, src2d, dst2d):
    spw = (src2d.shape[0] * src2d.shape[1]) // (NW * SUBE)
    return _sc_scatter_kernel(spw, g_hbm.shape[0])(g_hbm, src2d, dst2d)


def _deg_inputs(adj, adj2):
    pattern = jnp.concatenate(
        [jnp.ones((64,), jnp.float32), jnp.zeros((64,), jnp.float32)])
    g_deg = jnp.stack([pattern, 1.0 - pattern])
    g_deg = jnp.tile(g_deg[None], (RSPREAD, 1, 1)).reshape(2 * RSPREAD, D)

    spread = (jnp.arange(E, dtype=jnp.int32) % RSPREAD) * 2
    pad = EPAD2 - 2 * E
    src = jnp.concatenate([
        spread, spread + 1,
        (jnp.arange(pad, dtype=jnp.int32) % RSPREAD) * 2,
    ]).reshape(EPAD2 // SUBE, SUBE)
    dst = jnp.concatenate([
        adj[1], adj2[1], N + (jnp.arange(pad, dtype=jnp.int32) % 8),
    ]).reshape(EPAD2 // SUBE, SUBE)
    return g_deg, src, dst


BR = 400
GRID = N // BR

_f32 = jnp.float32


def _t1_body(x_ref, w_ref, sd_ref,
             hw_ref, g_ref, dinv1_ref, inv1_ref, dinv2_ref, inv2_ref):
    d1 = 1.0 + sd_ref[0, :, 0:1] + sd_ref[1, :, 0:1]
    d2 = 1.0 + sd_ref[0, :, 64:65] + sd_ref[1, :, 64:65]
    dinv1 = lax.rsqrt(d1)
    dinv2 = lax.rsqrt(d2)
    inv1 = 1.0 / d1
    inv2 = 1.0 / d2
    hw = jnp.dot(x_ref[...], w_ref[...], preferred_element_type=_f32)
    hw_ref[...] = hw
    g_ref[...] = hw * dinv1
    dinv1_ref[...] = dinv1
    inv1_ref[...] = inv1
    dinv2_ref[...] = dinv2
    inv2_ref[...] = inv2


def _t1(x, W0, sdeg):
    return pl.pallas_call(
        _t1_body,
        grid=(GRID,),
        in_specs=[
            pl.BlockSpec((BR, D), lambda i: (i, 0)),
            pl.BlockSpec((D, D), lambda i: (0, 0)),
            pl.BlockSpec((2, BR, D), lambda i: (0, i, 0)),
        ],
        out_specs=[
            pl.BlockSpec((BR, D), lambda i: (i, 0)),
            pl.BlockSpec((BR, D), lambda i: (i, 0)),
            pl.BlockSpec((BR, 1), lambda i: (i, 0)),
            pl.BlockSpec((BR, 1), lambda i: (i, 0)),
            pl.BlockSpec((BR, 1), lambda i: (i, 0)),
            pl.BlockSpec((BR, 1), lambda i: (i, 0)),
        ],
        out_shape=[
            jax.ShapeDtypeStruct((N, D), _f32),
            jax.ShapeDtypeStruct((N, D), _f32),
            jax.ShapeDtypeStruct((N, 1), _f32),
            jax.ShapeDtypeStruct((N, 1), _f32),
            jax.ShapeDtypeStruct((N, 1), _f32),
            jax.ShapeDtypeStruct((N, 1), _f32),
        ],
    )(x, W0, sdeg)


def _t2_body(s_ref, hw_ref, dinv1_ref, inv1_ref, b_ref, w_ref, dinv2_ref,
             hw1_ref, g1_ref):
    conv = (dinv1_ref[...] * (s_ref[0] + s_ref[1])
            + hw_ref[...] * inv1_ref[...] + b_ref[...])
    h1 = jnp.maximum(conv, 0.0)
    hw1 = jnp.dot(h1, w_ref[...], preferred_element_type=_f32)
    hw1_ref[...] = hw1
    g1_ref[...] = hw1 * dinv2_ref[...]


def _t2(S, hW, dinv1, inv1, b0, W1, dinv2):
    return pl.pallas_call(
        _t2_body,
        grid=(GRID,),
        in_specs=[
            pl.BlockSpec((2, BR, D), lambda i: (0, i, 0)),
            pl.BlockSpec((BR, D), lambda i: (i, 0)),
            pl.BlockSpec((BR, 1), lambda i: (i, 0)),
            pl.BlockSpec((BR, 1), lambda i: (i, 0)),
            pl.BlockSpec((1, D), lambda i: (0, 0)),
            pl.BlockSpec((D, D), lambda i: (0, 0)),
            pl.BlockSpec((BR, 1), lambda i: (i, 0)),
        ],
        out_specs=[
            pl.BlockSpec((BR, D), lambda i: (i, 0)),
            pl.BlockSpec((BR, D), lambda i: (i, 0)),
        ],
        out_shape=[
            jax.ShapeDtypeStruct((N, D), _f32),
            jax.ShapeDtypeStruct((N, D), _f32),
        ],
    )(S, hW, dinv1, inv1, b0, W1, dinv2)


def _t3_body(s_ref, hw1_ref, dinv2_ref, inv2_ref, b1_ref,
             wl1_ref, bl1_ref, wl2_ref, bl2_ref, w4_ref,
             hw4_ref, g4_ref):
    conv = (dinv2_ref[...] * (s_ref[0] + s_ref[1])
            + hw1_ref[...] * inv2_ref[...] + b1_ref[...])
    h2 = jnp.maximum(conv, 0.0)
    h3 = jnp.maximum(jnp.dot(h2, wl1_ref[...], preferred_element_type=_f32)
                     + bl1_ref[...], 0.0)
    h4 = jnp.maximum(jnp.dot(h3, wl2_ref[...], preferred_element_type=_f32)
                     + bl2_ref[...], 0.0)
    hw4 = jnp.dot(h4, w4_ref[...], preferred_element_type=_f32)
    hw4_ref[...] = hw4
    g4_ref[...] = hw4 * dinv2_ref[...]


def _t3(S, hW1, dinv2, inv2, b1, Wl1, bl1, Wl2, bl2, W4):
    return pl.pallas_call(
        _t3_body,
        grid=(GRID,),
        in_specs=[
            pl.BlockSpec((2, BR, D), lambda i: (0, i, 0)),
            pl.BlockSpec((BR, D), lambda i: (i, 0)),
            pl.BlockSpec((BR, 1), lambda i: (i, 0)),
            pl.BlockSpec((BR, 1), lambda i: (i, 0)),
            pl.BlockSpec((1, D), lambda i: (0, 0)),
            pl.BlockSpec((D, D), lambda i: (0, 0)),
            pl.BlockSpec((1, D), lambda i: (0, 0)),
            pl.BlockSpec((D, D), lambda i: (0, 0)),
            pl.BlockSpec((1, D), lambda i: (0, 0)),
            pl.BlockSpec((D, D), lambda i: (0, 0)),
        ],
        out_specs=[
            pl.BlockSpec((BR, D), lambda i: (i, 0)),
            pl.BlockSpec((BR, D), lambda i: (i, 0)),
        ],
        out_shape=[
            jax.ShapeDtypeStruct((N, D), _f32),
            jax.ShapeDtypeStruct((N, D), _f32),
        ],
    )(S, hW1, dinv2, inv2, b1, Wl1, bl1, Wl2, bl2, W4)


def _t4_body(s_ref, hw4_ref, dinv2_ref, inv2_ref, b4_ref, mode_ref, out_ref):
    o = (dinv2_ref[...] * (s_ref[0] + s_ref[1])
         + hw4_ref[...] * inv2_ref[...] + b4_ref[...])
    m = o - jnp.max(o, axis=1, keepdims=True)
    e = jnp.exp(m)
    sm = e / jnp.sum(e, axis=1, keepdims=True)
    out_ref[...] = jnp.where(mode_ref[...] == 1, o, sm)


def _t4(S, hW4, dinv2, inv2, b4, mode_arr):
    return pl.pallas_call(
        _t4_body,
        grid=(GRID,),
        in_specs=[
            pl.BlockSpec((2, BR, D), lambda i: (0, i, 0)),
            pl.BlockSpec((BR, D), lambda i: (i, 0)),
            pl.BlockSpec((BR, 1), lambda i: (i, 0)),
            pl.BlockSpec((BR, 1), lambda i: (i, 0)),
            pl.BlockSpec((1, D), lambda i: (0, 0)),
            pl.BlockSpec((1, 1), lambda i: (0, 0)),
        ],
        out_specs=pl.BlockSpec((BR, D), lambda i: (i, 0)),
        out_shape=jax.ShapeDtypeStruct((N, D), _f32),
    )(S, hW4, dinv2, inv2, b4, mode_arr)


def _pad_edges(src, dst):
    pad = EPAD - E
    srcp = jnp.concatenate(
        [src, jnp.zeros((pad,), jnp.int32)]).reshape(EPAD // SUBE, SUBE)
    dstp = jnp.concatenate(
        [dst, N + (jnp.arange(pad, dtype=jnp.int32) % 8)]).reshape(EPAD // SUBE, SUBE)
    return srcp, dstp


def kernel(x, adj, adj2, mode, W0, b0, W1, b1, Wl1, bl1, Wl2, bl2, W4, b4):
    src1, dst1 = _pad_edges(adj[0], adj[1])
    src2, dst2 = _pad_edges(adj2[0], adj2[1])

    g_deg, dsrc, ddst = _deg_inputs(adj, adj2)
    sdeg = _sc_scatter(g_deg, dsrc, ddst).reshape(2, N, D)

    hW0, g0, dinv1, inv1, dinv2, inv2 = _t1(x, W0, sdeg)

    S0 = _sc_scatter(g0, src1, dst1).reshape(2, N, D)
    hW1, g1 = _t2(S0, hW0, dinv1, inv1, b0.reshape(1, D), W1, dinv2)

    S1 = _sc_scatter(g1, src2, dst2).reshape(2, N, D)
    hW4, g4 = _t3(S1, hW1, dinv2, inv2, b1.reshape(1, D),
                  Wl1, bl1.reshape(1, D), Wl2, bl2.reshape(1, D), W4)

    S4 = _sc_scatter(g4, src2, dst2).reshape(2, N, D)
    mode_arr = jnp.asarray(mode, jnp.int32).reshape(1, 1)
    return _t4(S4, hW4, dinv2, inv2, b4.reshape(1, D), mode_arr)


# speedup vs baseline: 1.0327x; 1.0327x over previous
import functools

import jax
import jax.numpy as jnp
from jax import lax
from jax.experimental import pallas as pl
from jax.experimental.pallas import tpu as pltpu
from jax.experimental.pallas import tpu_sc as plsc

N = 10000
E = 320000
D = 128

NC = 2
NS = 16
NW = NC * NS

SUB = 128
SUBE = 64
GRP = 16
EPAD = 327680
EPAD2 = 655360
RSPREAD = 512

NA = N + 8
RT0 = 624
RT15 = NA - 15 * RT0


@functools.cache
def _mesh():
    return plsc.VectorSubcoreMesh(core_axis_name="c", subcore_axis_name="s")


def _zero_rows(ref, nrows, width):
    def body(i, carry):
        for j in range(width // 16):
            ref[i, pl.ds(j * 16, 16)] = jnp.zeros((16,), jnp.float32)
        return carry
    lax.fori_loop(0, nrows, body, 0)


def _fill_ones(ref, nrows, width):
    def body(i, carry):
        for j in range(width // 16):
            ref[i, pl.ds(j * 16, 16)] = jnp.ones((16,), jnp.float32)
        return carry
    lax.fori_loop(0, nrows, body, 0)


def _tile_rows(s):
    r0 = s * RT0
    cnt_is_last = s == NS - 1
    return r0, cnt_is_last


def _zero_acc_slice(acc, zb, s, width):
    r0 = s * RT0

    def body(j, carry):
        pltpu.sync_copy(zb, acc.at[pl.ds(r0 + j * SUBE, SUBE)])
        return carry
    lax.fori_loop(0, 9, body, 0)

    @pl.when(s < NS - 1)
    def _():
        pltpu.sync_copy(zb.at[pl.ds(0, 48)], acc.at[pl.ds(r0 + 576, 48)])

    @pl.when(s == NS - 1)
    def _():
        pltpu.sync_copy(zb, acc.at[pl.ds(r0 + 576, SUBE)])
        pltpu.sync_copy(zb.at[pl.ds(0, 8)], acc.at[pl.ds(r0 + 640, 8)])


def _export_acc_slice(acc, stage, out, c, s):
    r0 = s * RT0
    base = c * N + r0

    def body(j, carry):
        pltpu.sync_copy(acc.at[pl.ds(r0 + j * SUBE, SUBE)], stage)
        pltpu.sync_copy(stage, out.at[pl.ds(base + j * SUBE, SUBE)])
        return carry
    lax.fori_loop(0, 9, body, 0)

    @pl.when(s < NS - 1)
    def _():
        pltpu.sync_copy(acc.at[pl.ds(r0 + 576, 48)], stage.at[pl.ds(0, 48)])
        pltpu.sync_copy(stage.at[pl.ds(0, 48)], out.at[pl.ds(base + 576, 48)])

    @pl.when(s == NS - 1)
    def _():
        pltpu.sync_copy(acc.at[pl.ds(r0 + 576, SUBE)], stage)
        pltpu.sync_copy(stage, out.at[pl.ds(base + 576, SUBE)])


NBUF = 4


def _make_scatter_body(spw):
    ngrp = spw // GRP

    def _sc_scatter_body(g_hbm, src2d, dst2d, out,
                         isa, ida, isb, idb, b0, b1, b2, b3, acc,
                         g0s, g1s, g2s, g3s, c0s, c1s, c2s, c3s, ps, pd):
        c = lax.axis_index("c")
        s = lax.axis_index("s")
        wid = c * NS + s

        _zero_rows(b0, SUBE, D)
        _zero_acc_slice(acc, b0, s, D)
        plsc.subcore_barrier()

        bufs = (b0, b1, b2, b3)
        gsem = (g0s, g1s, g2s, g3s)
        csem = (c0s, c1s, c2s, c3s)

        def group(j, cur, nxt, prefetch):
            csi, cdi = cur
            nsi, ndi = nxt
            pdescs = []
            if prefetch:
                nb = wid * spw + (j + 1) * GRP
                pdescs.append(
                    pltpu.async_copy(src2d.at[pl.ds(nb, GRP)], nsi, ps))
                pdescs.append(
                    pltpu.async_copy(dst2d.at[pl.ds(nb, GRP)], ndi, pd))
            gd = [None] * GRP
            sd = [None] * GRP
            for b in range(NBUF):
                gd[b] = pltpu.async_copy(g_hbm.at[csi.at[b]], bufs[b], gsem[b])
            for k in range(GRP):
                b = k % NBUF
                gd[k].wait()
                sd[k] = pltpu.async_copy(bufs[b], acc.at[cdi.at[k]],
                                         csem[b], add=True)
                if 2 <= k < GRP - 2:
                    sd[k - 2].wait()
                    b2_ = (k + 2) % NBUF
                    gd[k + 2] = pltpu.async_copy(
                        g_hbm.at[csi.at[k + 2]], bufs[b2_], gsem[b2_])
            for k in range(GRP - NBUF, GRP):
                sd[k].wait()
            for d in pdescs:
                d.wait()

        gb0 = wid * spw
        pltpu.sync_copy(src2d.at[pl.ds(gb0, GRP)], isa)
        pltpu.sync_copy(dst2d.at[pl.ds(gb0, GRP)], ida)

        ia = (isa, ida)
        ib = (isb, idb)
        group(0, ia, ib, True)

        def pair(p, carry):
            group(2 * p + 1, ib, ia, True)
            group(2 * p + 2, ia, ib, True)
            return carry
        lax.fori_loop(0, (ngrp - 2) // 2, pair, 0)

        group(ngrp - 1, ib, ia, False)

        plsc.subcore_barrier()
        _export_acc_slice(acc, b0, out, c, s)

    return _sc_scatter_body


@functools.cache
def _sc_scatter_kernel(spw, nrows):
    return pl.kernel(
        _make_scatter_body(spw),
        out_type=jax.ShapeDtypeStruct((2 * N, D), jnp.float32),
        mesh=_mesh(),
        scratch_types=[
            pltpu.VMEM((GRP, SUBE), jnp.int32),
            pltpu.VMEM((GRP, SUBE), jnp.int32),
            pltpu.VMEM((GRP, SUBE), jnp.int32),
            pltpu.VMEM((GRP, SUBE), jnp.int32),
            pltpu.VMEM((SUBE, D), jnp.float32),
            pltpu.VMEM((SUBE, D), jnp.float32),
            pltpu.VMEM((SUBE, D), jnp.float32),
            pltpu.VMEM((SUBE, D), jnp.float32),
            pltpu.VMEM_SHARED((NA, D), jnp.float32),
            pltpu.SemaphoreType.DMA,
            pltpu.SemaphoreType.DMA,
            pltpu.SemaphoreType.DMA,
            pltpu.SemaphoreType.DMA,
            pltpu.SemaphoreType.DMA,
            pltpu.SemaphoreType.DMA,
            pltpu.SemaphoreType.DMA,
            pltpu.SemaphoreType.DMA,
            pltpu.SemaphoreType.DMA,
            pltpu.SemaphoreType.DMA,
        ],
    )


def _sc_scatter(g_hbm, src2d, dst2d):
    spw = (src2d.shape[0] * src2d.shape[1]) // (NW * SUBE)
    return _sc_scatter_kernel(spw, g_hbm.shape[0])(g_hbm, src2d, dst2d)


def _deg_inputs(adj, adj2):
    pattern = jnp.concatenate(
        [jnp.ones((64,), jnp.float32), jnp.zeros((64,), jnp.float32)])
    g_deg = jnp.stack([pattern, 1.0 - pattern])
    g_deg = jnp.tile(g_deg[None], (RSPREAD, 1, 1)).reshape(2 * RSPREAD, D)

    spread = (jnp.arange(E, dtype=jnp.int32) % RSPREAD) * 2
    pad = EPAD2 - 2 * E
    src = jnp.concatenate([
        spread, spread + 1,
        (jnp.arange(pad, dtype=jnp.int32) % RSPREAD) * 2,
    ]).reshape(EPAD2 // SUBE, SUBE)
    dst = jnp.concatenate([
        adj[1], adj2[1], N + (jnp.arange(pad, dtype=jnp.int32) % 8),
    ]).reshape(EPAD2 // SUBE, SUBE)
    return g_deg, src, dst


BR = 400
GRID = N // BR

_f32 = jnp.float32


def _t1_body(x_ref, w_ref, sd_ref,
             hw_ref, g_ref, dinv1_ref, inv1_ref, dinv2_ref, inv2_ref):
    d1 = 1.0 + sd_ref[0, :, 0:1] + sd_ref[1, :, 0:1]
    d2 = 1.0 + sd_ref[0, :, 64:65] + sd_ref[1, :, 64:65]
    dinv1 = lax.rsqrt(d1)
    dinv2 = lax.rsqrt(d2)
    inv1 = 1.0 / d1
    inv2 = 1.0 / d2
    hw = jnp.dot(x_ref[...], w_ref[...], preferred_element_type=_f32)
    hw_ref[...] = hw
    g_ref[...] = hw * dinv1
    dinv1_ref[...] = dinv1
    inv1_ref[...] = inv1
    dinv2_ref[...] = dinv2
    inv2_ref[...] = inv2


def _t1(x, W0, sdeg):
    return pl.pallas_call(
        _t1_body,
        grid=(GRID,),
        in_specs=[
            pl.BlockSpec((BR, D), lambda i: (i, 0)),
            pl.BlockSpec((D, D), lambda i: (0, 0)),
            pl.BlockSpec((2, BR, D), lambda i: (0, i, 0)),
        ],
        out_specs=[
            pl.BlockSpec((BR, D), lambda i: (i, 0)),
            pl.BlockSpec((BR, D), lambda i: (i, 0)),
            pl.BlockSpec((BR, 1), lambda i: (i, 0)),
            pl.BlockSpec((BR, 1), lambda i: (i, 0)),
            pl.BlockSpec((BR, 1), lambda i: (i, 0)),
            pl.BlockSpec((BR, 1), lambda i: (i, 0)),
        ],
        out_shape=[
            jax.ShapeDtypeStruct((N, D), _f32),
            jax.ShapeDtypeStruct((N, D), _f32),
            jax.ShapeDtypeStruct((N, 1), _f32),
            jax.ShapeDtypeStruct((N, 1), _f32),
            jax.ShapeDtypeStruct((N, 1), _f32),
            jax.ShapeDtypeStruct((N, 1), _f32),
        ],
    )(x, W0, sdeg)


def _t2_body(s_ref, hw_ref, dinv1_ref, inv1_ref, b_ref, w_ref, dinv2_ref,
             hw1_ref, g1_ref):
    conv = (dinv1_ref[...] * (s_ref[0] + s_ref[1])
            + hw_ref[...] * inv1_ref[...] + b_ref[...])
    h1 = jnp.maximum(conv, 0.0)
    hw1 = jnp.dot(h1, w_ref[...], preferred_element_type=_f32)
    hw1_ref[...] = hw1
    g1_ref[...] = hw1 * dinv2_ref[...]


def _t2(S, hW, dinv1, inv1, b0, W1, dinv2):
    return pl.pallas_call(
        _t2_body,
        grid=(GRID,),
        in_specs=[
            pl.BlockSpec((2, BR, D), lambda i: (0, i, 0)),
            pl.BlockSpec((BR, D), lambda i: (i, 0)),
            pl.BlockSpec((BR, 1), lambda i: (i, 0)),
            pl.BlockSpec((BR, 1), lambda i: (i, 0)),
            pl.BlockSpec((1, D), lambda i: (0, 0)),
            pl.BlockSpec((D, D), lambda i: (0, 0)),
            pl.BlockSpec((BR, 1), lambda i: (i, 0)),
        ],
        out_specs=[
            pl.BlockSpec((BR, D), lambda i: (i, 0)),
            pl.BlockSpec((BR, D), lambda i: (i, 0)),
        ],
        out_shape=[
            jax.ShapeDtypeStruct((N, D), _f32),
            jax.ShapeDtypeStruct((N, D), _f32),
        ],
    )(S, hW, dinv1, inv1, b0, W1, dinv2)


def _t3_body(s_ref, hw1_ref, dinv2_ref, inv2_ref, b1_ref,
             wl1_ref, bl1_ref, wl2_ref, bl2_ref, w4_ref,
             hw4_ref, g4_ref):
    conv = (dinv2_ref[...] * (s_ref[0] + s_ref[1])
            + hw1_ref[...] * inv2_ref[...] + b1_ref[...])
    h2 = jnp.maximum(conv, 0.0)
    h3 = jnp.maximum(jnp.dot(h2, wl1_ref[...], preferred_element_type=_f32)
                     + bl1_ref[...], 0.0)
    h4 = jnp.maximum(jnp.dot(h3, wl2_ref[...], preferred_element_type=_f32)
                     + bl2_ref[...], 0.0)
    hw4 = jnp.dot(h4, w4_ref[...], preferred_element_type=_f32)
    hw4_ref[...] = hw4
    g4_ref[...] = hw4 * dinv2_ref[...]


def _t3(S, hW1, dinv2, inv2, b1, Wl1, bl1, Wl2, bl2, W4):
    return pl.pallas_call(
        _t3_body,
        grid=(GRID,),
        in_specs=[
            pl.BlockSpec((2, BR, D), lambda i: (0, i, 0)),
            pl.BlockSpec((BR, D), lambda i: (i, 0)),
            pl.BlockSpec((BR, 1), lambda i: (i, 0)),
            pl.BlockSpec((BR, 1), lambda i: (i, 0)),
            pl.BlockSpec((1, D), lambda i: (0, 0)),
            pl.BlockSpec((D, D), lambda i: (0, 0)),
            pl.BlockSpec((1, D), lambda i: (0, 0)),
            pl.BlockSpec((D, D), lambda i: (0, 0)),
            pl.BlockSpec((1, D), lambda i: (0, 0)),
            pl.BlockSpec((D, D), lambda i: (0, 0)),
        ],
        out_specs=[
            pl.BlockSpec((BR, D), lambda i: (i, 0)),
            pl.BlockSpec((BR, D), lambda i: (i, 0)),
        ],
        out_shape=[
            jax.ShapeDtypeStruct((N, D), _f32),
            jax.ShapeDtypeStruct((N, D), _f32),
        ],
    )(S, hW1, dinv2, inv2, b1, Wl1, bl1, Wl2, bl2, W4)


def _t4_body(s_ref, hw4_ref, dinv2_ref, inv2_ref, b4_ref, mode_ref, out_ref):
    o = (dinv2_ref[...] * (s_ref[0] + s_ref[1])
         + hw4_ref[...] * inv2_ref[...] + b4_ref[...])
    m = o - jnp.max(o, axis=1, keepdims=True)
    e = jnp.exp(m)
    sm = e / jnp.sum(e, axis=1, keepdims=True)
    out_ref[...] = jnp.where(mode_ref[...] == 1, o, sm)


def _t4(S, hW4, dinv2, inv2, b4, mode_arr):
    return pl.pallas_call(
        _t4_body,
        grid=(GRID,),
        in_specs=[
            pl.BlockSpec((2, BR, D), lambda i: (0, i, 0)),
            pl.BlockSpec((BR, D), lambda i: (i, 0)),
            pl.BlockSpec((BR, 1), lambda i: (i, 0)),
            pl.BlockSpec((BR, 1), lambda i: (i, 0)),
            pl.BlockSpec((1, D), lambda i: (0, 0)),
            pl.BlockSpec((1, 1), lambda i: (0, 0)),
        ],
        out_specs=pl.BlockSpec((BR, D), lambda i: (i, 0)),
        out_shape=jax.ShapeDtypeStruct((N, D), _f32),
    )(S, hW4, dinv2, inv2, b4, mode_arr)


def _pad_edges(src, dst):
    pad = EPAD - E
    srcp = jnp.concatenate(
        [src, jnp.zeros((pad,), jnp.int32)]).reshape(EPAD // SUBE, SUBE)
    dstp = jnp.concatenate(
        [dst, N + (jnp.arange(pad, dtype=jnp.int32) % 8)]).reshape(EPAD // SUBE, SUBE)
    return srcp, dstp


def kernel(x, adj, adj2, mode, W0, b0, W1, b1, Wl1, bl1, Wl2, bl2, W4, b4):
    src1, dst1 = _pad_edges(adj[0], adj[1])
    src2, dst2 = _pad_edges(adj2[0], adj2[1])

    g_deg, dsrc, ddst = _deg_inputs(adj, adj2)
    sdeg = _sc_scatter(g_deg, dsrc, ddst).reshape(2, N, D)

    hW0, g0, dinv1, inv1, dinv2, inv2 = _t1(x, W0, sdeg)

    S0 = _sc_scatter(g0, src1, dst1).reshape(2, N, D)
    hW1, g1 = _t2(S0, hW0, dinv1, inv1, b0.reshape(1, D), W1, dinv2)

    S1 = _sc_scatter(g1, src2, dst2).reshape(2, N, D)
    hW4, g4 = _t3(S1, hW1, dinv2, inv2, b1.reshape(1, D),
                  Wl1, bl1.reshape(1, D), Wl2, bl2.reshape(1, D), W4)

    S4 = _sc_scatter(g4, src2, dst2).reshape(2, N, D)
    mode_arr = jnp.asarray(mode, jnp.int32).reshape(1, 1)
    return _t4(S4, hW4, dinv2, inv2, b4.reshape(1, D), mode_arr)


# speedup vs baseline: 1.1899x; 1.1522x over previous
import functools

import jax
import jax.numpy as jnp
from jax import lax
from jax.experimental import pallas as pl
from jax.experimental.pallas import tpu as pltpu
from jax.experimental.pallas import tpu_sc as plsc

N = 10000
E = 320000
D = 128

NC = 2
NS = 16
NW = NC * NS

SUB = 128
GRP = 8
SPW = 80
EPAD = NW * SPW * SUB
SPW2 = 160
EPAD2 = NW * SPW2 * SUB
RSPREAD = 512

NA = N + 8
RT0 = 624
RT15 = NA - 15 * RT0


@functools.cache
def _mesh():
    return plsc.VectorSubcoreMesh(core_axis_name="c", subcore_axis_name="s")


def _zero_rows(ref, nrows, width):
    def body(i, carry):
        for j in range(width // 16):
            ref[i, pl.ds(j * 16, 16)] = jnp.zeros((16,), jnp.float32)
        return carry
    lax.fori_loop(0, nrows, body, 0)


def _fill_ones(ref, nrows, width):
    def body(i, carry):
        for j in range(width // 16):
            ref[i, pl.ds(j * 16, 16)] = jnp.ones((16,), jnp.float32)
        return carry
    lax.fori_loop(0, nrows, body, 0)


def _tile_rows(s):
    r0 = s * RT0
    cnt_is_last = s == NS - 1
    return r0, cnt_is_last


def _zero_acc_slice(acc, zb, s, width):
    r0 = s * RT0
    def body(j, carry):
        pltpu.sync_copy(zb, acc.at[pl.ds(r0 + j * SUB, SUB)])
        return carry
    lax.fori_loop(0, 4, body, 0)

    @pl.when(s < NS - 1)
    def _():
        pltpu.sync_copy(zb.at[pl.ds(0, 112)], acc.at[pl.ds(r0 + 512, 112)])

    @pl.when(s == NS - 1)
    def _():
        pltpu.sync_copy(zb, acc.at[pl.ds(r0 + 512, SUB)])
        pltpu.sync_copy(zb.at[pl.ds(0, 8)], acc.at[pl.ds(r0 + 640, 8)])


def _export_acc_slice(acc, stage, out, c, s):
    r0 = s * RT0
    base = c * N + r0

    def body(j, carry):
        pltpu.sync_copy(acc.at[pl.ds(r0 + j * SUB, SUB)], stage)
        pltpu.sync_copy(stage, out.at[pl.ds(base + j * SUB, SUB)])
        return carry
    lax.fori_loop(0, 4, body, 0)

    @pl.when(s < NS - 1)
    def _():
        pltpu.sync_copy(acc.at[pl.ds(r0 + 512, 112)], stage.at[pl.ds(0, 112)])
        pltpu.sync_copy(stage.at[pl.ds(0, 112)], out.at[pl.ds(base + 512, 112)])

    @pl.when(s == NS - 1)
    def _():
        pltpu.sync_copy(acc.at[pl.ds(r0 + 512, SUB)], stage)
        pltpu.sync_copy(stage, out.at[pl.ds(base + 512, SUB)])


def _make_scatter_body(spw0, spw1):
    ngrp0 = spw0 // GRP
    ngrp1 = spw1 // GRP

    def _sc_scatter_body(g_hbm, src2d, dst2d, out, sidx, didx, ga, gb, acc,
                         sema, semb):
        c = lax.axis_index("c")
        s = lax.axis_index("s")

        _zero_rows(ga, SUB, D)
        _zero_acc_slice(acc, ga, s, D)
        plsc.subcore_barrier()

        bufs = (ga, gb)
        sems = (sema, semb)

        base_c = jnp.where(c == 0, s * spw0, NS * spw0 + s * spw1)
        ngrp_c = jnp.where(c == 0, ngrp0, ngrp1)

        def body(j, carry):
            g0 = base_c + j * GRP
            pltpu.sync_copy(src2d.at[pl.ds(g0, GRP)], sidx)
            pltpu.sync_copy(dst2d.at[pl.ds(g0, GRP)], didx)
            descs = [None, None]
            descs[0] = pltpu.async_copy(g_hbm.at[sidx.at[0]], bufs[0], sems[0])
            for k in range(GRP):
                cur = k % 2
                nxt = 1 - cur
                if k + 1 < GRP:
                    descs[nxt] = pltpu.async_copy(
                        g_hbm.at[sidx.at[k + 1]], bufs[nxt], sems[nxt])
                descs[cur].wait()
                pltpu.sync_copy(bufs[cur], acc.at[didx.at[k]], add=True)
            return carry
        lax.fori_loop(0, ngrp_c, body, 0)

        plsc.subcore_barrier()
        _export_acc_slice(acc, ga, out, c, s)

    return _sc_scatter_body


@functools.cache
def _sc_scatter_kernel(spw0, spw1, nrows):
    return pl.kernel(
        _make_scatter_body(spw0, spw1),
        out_type=jax.ShapeDtypeStruct((2 * N, D), jnp.float32),
        mesh=_mesh(),
        scratch_types=[
            pltpu.VMEM((GRP, SUB), jnp.int32),
            pltpu.VMEM((GRP, SUB), jnp.int32),
            pltpu.VMEM((SUB, D), jnp.float32),
            pltpu.VMEM((SUB, D), jnp.float32),
            pltpu.VMEM_SHARED((NA, D), jnp.float32),
            pltpu.SemaphoreType.DMA,
            pltpu.SemaphoreType.DMA,
        ],
    )


FAST_FRAC_NUM, FAST_FRAC_DEN = 3, 4


def _sc_scatter(g_hbm, src2d, dst2d, balanced=False):
    spw_tot = (src2d.shape[0] * src2d.shape[1]) // (NW * SUB)
    if balanced:
        spw0 = spw1 = spw_tot
    else:
        spw0 = (2 * spw_tot * FAST_FRAC_NUM // FAST_FRAC_DEN) // GRP * GRP
        spw1 = 2 * spw_tot - spw0
    return _sc_scatter_kernel(spw0, spw1, g_hbm.shape[0])(g_hbm, src2d, dst2d)


def _deg_inputs(adj, adj2):
    pattern = jnp.concatenate(
        [jnp.ones((64,), jnp.float32), jnp.zeros((64,), jnp.float32)])
    g_deg = jnp.stack([pattern, 1.0 - pattern])
    g_deg = jnp.tile(g_deg[None], (RSPREAD, 1, 1)).reshape(2 * RSPREAD, D)

    spread = (jnp.arange(E, dtype=jnp.int32) % RSPREAD) * 2
    pad = EPAD2 - 2 * E
    src = jnp.concatenate([
        spread, spread + 1,
        (jnp.arange(pad, dtype=jnp.int32) % RSPREAD) * 2,
    ]).reshape(EPAD2 // SUB, SUB)
    dst = jnp.concatenate([
        adj[1], adj2[1], N + (jnp.arange(pad, dtype=jnp.int32) % 8),
    ]).reshape(EPAD2 // SUB, SUB)
    return g_deg, src, dst


BR = 400
GRID = N // BR

_f32 = jnp.float32


def _t1_body(x_ref, w_ref, sd_ref,
             hw_ref, g_ref, dinv1_ref, inv1_ref, dinv2_ref, inv2_ref):
    d1 = 1.0 + sd_ref[0, :, 0:1] + sd_ref[1, :, 0:1]
    d2 = 1.0 + sd_ref[0, :, 64:65] + sd_ref[1, :, 64:65]
    dinv1 = lax.rsqrt(d1)
    dinv2 = lax.rsqrt(d2)
    inv1 = 1.0 / d1
    inv2 = 1.0 / d2
    hw = jnp.dot(x_ref[...], w_ref[...], preferred_element_type=_f32)
    hw_ref[...] = hw
    g_ref[...] = hw * dinv1
    dinv1_ref[...] = dinv1
    inv1_ref[...] = inv1
    dinv2_ref[...] = dinv2
    inv2_ref[...] = inv2


def _t1(x, W0, sdeg):
    return pl.pallas_call(
        _t1_body,
        grid=(GRID,),
        in_specs=[
            pl.BlockSpec((BR, D), lambda i: (i, 0)),
            pl.BlockSpec((D, D), lambda i: (0, 0)),
            pl.BlockSpec((2, BR, D), lambda i: (0, i, 0)),
        ],
        out_specs=[
            pl.BlockSpec((BR, D), lambda i: (i, 0)),
            pl.BlockSpec((BR, D), lambda i: (i, 0)),
            pl.BlockSpec((BR, 1), lambda i: (i, 0)),
            pl.BlockSpec((BR, 1), lambda i: (i, 0)),
            pl.BlockSpec((BR, 1), lambda i: (i, 0)),
            pl.BlockSpec((BR, 1), lambda i: (i, 0)),
        ],
        out_shape=[
            jax.ShapeDtypeStruct((N, D), _f32),
            jax.ShapeDtypeStruct((N, D), _f32),
            jax.ShapeDtypeStruct((N, 1), _f32),
            jax.ShapeDtypeStruct((N, 1), _f32),
            jax.ShapeDtypeStruct((N, 1), _f32),
            jax.ShapeDtypeStruct((N, 1), _f32),
        ],
    )(x, W0, sdeg)


def _t2_body(s_ref, hw_ref, dinv1_ref, inv1_ref, b_ref, w_ref, dinv2_ref,
             hw1_ref, g1_ref):
    conv = (dinv1_ref[...] * (s_ref[0] + s_ref[1])
            + hw_ref[...] * inv1_ref[...] + b_ref[...])
    h1 = jnp.maximum(conv, 0.0)
    hw1 = jnp.dot(h1, w_ref[...], preferred_element_type=_f32)
    hw1_ref[...] = hw1
    g1_ref[...] = hw1 * dinv2_ref[...]


def _t2(S, hW, dinv1, inv1, b0, W1, dinv2):
    return pl.pallas_call(
        _t2_body,
        grid=(GRID,),
        in_specs=[
            pl.BlockSpec((2, BR, D), lambda i: (0, i, 0)),
            pl.BlockSpec((BR, D), lambda i: (i, 0)),
            pl.BlockSpec((BR, 1), lambda i: (i, 0)),
            pl.BlockSpec((BR, 1), lambda i: (i, 0)),
            pl.BlockSpec((1, D), lambda i: (0, 0)),
            pl.BlockSpec((D, D), lambda i: (0, 0)),
            pl.BlockSpec((BR, 1), lambda i: (i, 0)),
        ],
        out_specs=[
            pl.BlockSpec((BR, D), lambda i: (i, 0)),
            pl.BlockSpec((BR, D), lambda i: (i, 0)),
        ],
        out_shape=[
            jax.ShapeDtypeStruct((N, D), _f32),
            jax.ShapeDtypeStruct((N, D), _f32),
        ],
    )(S, hW, dinv1, inv1, b0, W1, dinv2)


def _t3_body(s_ref, hw1_ref, dinv2_ref, inv2_ref, b1_ref,
             wl1_ref, bl1_ref, wl2_ref, bl2_ref, w4_ref,
             hw4_ref, g4_ref):
    conv = (dinv2_ref[...] * (s_ref[0] + s_ref[1])
            + hw1_ref[...] * inv2_ref[...] + b1_ref[...])
    h2 = jnp.maximum(conv, 0.0)
    h3 = jnp.maximum(jnp.dot(h2, wl1_ref[...], preferred_element_type=_f32)
                     + bl1_ref[...], 0.0)
    h4 = jnp.maximum(jnp.dot(h3, wl2_ref[...], preferred_element_type=_f32)
                     + bl2_ref[...], 0.0)
    hw4 = jnp.dot(h4, w4_ref[...], preferred_element_type=_f32)
    hw4_ref[...] = hw4
    g4_ref[...] = hw4 * dinv2_ref[...]


def _t3(S, hW1, dinv2, inv2, b1, Wl1, bl1, Wl2, bl2, W4):
    return pl.pallas_call(
        _t3_body,
        grid=(GRID,),
        in_specs=[
            pl.BlockSpec((2, BR, D), lambda i: (0, i, 0)),
            pl.BlockSpec((BR, D), lambda i: (i, 0)),
            pl.BlockSpec((BR, 1), lambda i: (i, 0)),
            pl.BlockSpec((BR, 1), lambda i: (i, 0)),
            pl.BlockSpec((1, D), lambda i: (0, 0)),
            pl.BlockSpec((D, D), lambda i: (0, 0)),
            pl.BlockSpec((1, D), lambda i: (0, 0)),
            pl.BlockSpec((D, D), lambda i: (0, 0)),
            pl.BlockSpec((1, D), lambda i: (0, 0)),
            pl.BlockSpec((D, D), lambda i: (0, 0)),
        ],
        out_specs=[
            pl.BlockSpec((BR, D), lambda i: (i, 0)),
            pl.BlockSpec((BR, D), lambda i: (i, 0)),
        ],
        out_shape=[
            jax.ShapeDtypeStruct((N, D), _f32),
            jax.ShapeDtypeStruct((N, D), _f32),
        ],
    )(S, hW1, dinv2, inv2, b1, Wl1, bl1, Wl2, bl2, W4)


def _t4_body(s_ref, hw4_ref, dinv2_ref, inv2_ref, b4_ref, mode_ref, out_ref):
    o = (dinv2_ref[...] * (s_ref[0] + s_ref[1])
         + hw4_ref[...] * inv2_ref[...] + b4_ref[...])
    m = o - jnp.max(o, axis=1, keepdims=True)
    e = jnp.exp(m)
    sm = e / jnp.sum(e, axis=1, keepdims=True)
    out_ref[...] = jnp.where(mode_ref[...] == 1, o, sm)


def _t4(S, hW4, dinv2, inv2, b4, mode_arr):
    return pl.pallas_call(
        _t4_body,
        grid=(GRID,),
        in_specs=[
            pl.BlockSpec((2, BR, D), lambda i: (0, i, 0)),
            pl.BlockSpec((BR, D), lambda i: (i, 0)),
            pl.BlockSpec((BR, 1), lambda i: (i, 0)),
            pl.BlockSpec((BR, 1), lambda i: (i, 0)),
            pl.BlockSpec((1, D), lambda i: (0, 0)),
            pl.BlockSpec((1, 1), lambda i: (0, 0)),
        ],
        out_specs=pl.BlockSpec((BR, D), lambda i: (i, 0)),
        out_shape=jax.ShapeDtypeStruct((N, D), _f32),
    )(S, hW4, dinv2, inv2, b4, mode_arr)


def _pad_edges(src, dst):
    pad = EPAD - E
    srcp = jnp.concatenate(
        [src, jnp.zeros((pad,), jnp.int32)]).reshape(EPAD // SUB, SUB)
    dstp = jnp.concatenate(
        [dst, N + (jnp.arange(pad, dtype=jnp.int32) % 8)]).reshape(EPAD // SUB, SUB)
    return srcp, dstp


def kernel(x, adj, adj2, mode, W0, b0, W1, b1, Wl1, bl1, Wl2, bl2, W4, b4):
    src1, dst1 = _pad_edges(adj[0], adj[1])
    src2, dst2 = _pad_edges(adj2[0], adj2[1])

    g_deg, dsrc, ddst = _deg_inputs(adj, adj2)
    sdeg = _sc_scatter(g_deg, dsrc, ddst, balanced=True).reshape(2, N, D)

    hW0, g0, dinv1, inv1, dinv2, inv2 = _t1(x, W0, sdeg)

    S0 = _sc_scatter(g0, src1, dst1).reshape(2, N, D)
    hW1, g1 = _t2(S0, hW0, dinv1, inv1, b0.reshape(1, D), W1, dinv2)

    S1 = _sc_scatter(g1, src2, dst2).reshape(2, N, D)
    hW4, g4 = _t3(S1, hW1, dinv2, inv2, b1.reshape(1, D),
                  Wl1, bl1.reshape(1, D), Wl2, bl2.reshape(1, D), W4)

    S4 = _sc_scatter(g4, src2, dst2).reshape(2, N, D)
    mode_arr = jnp.asarray(mode, jnp.int32).reshape(1, 1)
    return _t4(S4, hW4, dinv2, inv2, b4.reshape(1, D), mode_arr)
